# baseline (speedup 1.0000x reference)
import functools

import jax
import jax.numpy as jnp
from jax import lax
from jax.experimental import pallas as pl

_HI = lax.Precision.HIGHEST
_HG = lax.Precision.HIGH


def _fps_body(xyzp_ref, nx_ref, *, npoint):
    x = xyzp_ref[0]
    y = xyzp_ref[1]
    z = xyzp_ref[2]
    B, N = x.shape
    iota = lax.broadcasted_iota(jnp.int32, (B, N), 1)

    def body(i, carry):
        dist, far = carry
        oh = (iota == far).astype(jnp.float32)
        cx = jnp.sum(x * oh, axis=1, keepdims=True)
        cy = jnp.sum(y * oh, axis=1, keepdims=True)
        cz = jnp.sum(z * oh, axis=1, keepdims=True)
        cen = jnp.concatenate([cx, cy, cz], axis=1)
        nx_ref[:, pl.ds(i, 1), :] = cen[:, None, :]
        d = (x - cx) ** 2 + (y - cy) ** 2 + (z - cz) ** 2
        dist = jnp.minimum(dist, d)
        m = jnp.max(dist, axis=1, keepdims=True)
        far = jnp.min(jnp.where(dist == m, iota, N), axis=1, keepdims=True)
        return dist, far

    dist0 = jnp.full((B, N), 1e10, jnp.float32)
    far0 = jnp.zeros((B, 1), jnp.int32)
    lax.fori_loop(0, npoint, body, (dist0, far0))


def _fps(xyzp, npoint):
    _, B, N = xyzp.shape
    return pl.pallas_call(
        functools.partial(_fps_body, npoint=npoint),
        out_shape=jax.ShapeDtypeStruct((B, npoint, 3), jnp.float32),
    )(xyzp)


def _sa_body_nofeat(xyzp_ref, nx_ref, m_ref, r_ref, xyzr_ref,
                    w1x_ref, b1_ref, w2_ref, b2_ref, w3_ref, b3_ref,
                    out_ref, *, K, KC, r2):
    _sa_common(xyzp_ref, nx_ref, m_ref, r_ref, xyzr_ref, None,
               w1x_ref, None, b1_ref, w2_ref, b2_ref, w3_ref, b3_ref,
               out_ref, K=K, KC=KC, r2=r2)


def _sa_body_feat(xyzp_ref, nx_ref, m_ref, r_ref, xyzr_ref, feat_ref,
                  w1x_ref, w1f_ref, b1_ref, w2_ref, b2_ref, w3_ref, b3_ref,
                  out_ref, *, K, KC, r2):
    _sa_common(xyzp_ref, nx_ref, m_ref, r_ref, xyzr_ref, feat_ref,
               w1x_ref, w1f_ref, b1_ref, w2_ref, b2_ref, w3_ref, b3_ref,
               out_ref, K=K, KC=KC, r2=r2)


def _sa_common(xyzp_ref, nx_ref, m_ref, r_ref, xyzr_ref, feat_ref,
               w1x_ref, w1f_ref, b1_ref, w2_ref, b2_ref, w3_ref, b3_ref,
               out_ref, *, K, KC, r2):
    xyzp = xyzp_ref[0]
    nxt = nx_ref[0]
    TS = nxt.shape[0]
    N = xyzp.shape[1]
    cout = w3_ref.shape[1]

    sq = ((nxt[:, 0:1] - xyzp[0:1, :]) ** 2
          + (nxt[:, 1:2] - xyzp[1:2, :]) ** 2
          + (nxt[:, 2:3] - xyzp[2:3, :]) ** 2)
    mask = sq <= r2
    maskb = mask.astype(jnp.bfloat16)
    cnt = jnp.dot(maskb, m_ref[...],
                  preferred_element_type=jnp.float32)
    cntm = cnt * mask.astype(jnp.float32)
    xyzr = xyzr_ref[0]
    xhi = xyzr.astype(jnp.bfloat16)
    xlo = (xyzr - xhi.astype(jnp.float32)).astype(jnp.bfloat16)
    nhi = nxt.astype(jnp.bfloat16)
    nlo = (nxt - nhi.astype(jnp.float32)).astype(jnp.bfloat16)
    rb = r_ref[...]
    cen = (jnp.dot(rb, nhi, preferred_element_type=jnp.float32)
           + jnp.dot(rb, nlo, preferred_element_type=jnp.float32))
    ones1n = jnp.ones((1, N), jnp.bfloat16)
    totalt = lax.dot_general(ones1n, maskb, (((1,), (1,)), ((), ())),
                             preferred_element_type=jnp.float32)

    kval3 = (lax.broadcasted_iota(jnp.int32, (KC, 1, 1), 0)
             .astype(jnp.float32) + 1.0)
    kval2 = lax.broadcasted_iota(jnp.int32, (KC, TS), 0).astype(jnp.float32)

    def chunk(c, acc):
        base = c.astype(jnp.float32) * KC
        csh = cntm - base
        sel = csh[None, :, :] == kval3
        g = sel.astype(jnp.bfloat16).reshape(KC * TS, N)
        gx = (jnp.dot(g, xhi, preferred_element_type=jnp.float32)
              + jnp.dot(g, xlo, preferred_element_type=jnp.float32))
        x0 = gx - cen
        h = jnp.dot(x0, w1x_ref[...])
        if feat_ref is not None:
            gf = jnp.dot(g, feat_ref[0].astype(jnp.bfloat16),
                         preferred_element_type=jnp.float32)
            h = h + jnp.dot(gf, w1f_ref[...])
        h = jnp.maximum(h + b1_ref[...], 0.0)
        h = jnp.maximum(jnp.dot(h, w2_ref[...]) + b2_ref[...], 0.0)
        h = jnp.maximum(jnp.dot(h, w3_ref[...]) + b3_ref[...], 0.0)
        penalty = jnp.where((kval2 + base) < totalt, 0.0, -1e30)
        hm = h.reshape(KC, TS, cout) + penalty[:, :, None]
        return jnp.maximum(acc, jnp.max(hm, axis=0))

    acc0 = jnp.full((TS, cout), -1e30, jnp.float32)
    out_ref[0] = lax.fori_loop(0, K // KC, chunk, acc0)


def _fold_bn(lyr):
    a = lyr['gamma'] * lax.rsqrt(lyr['var'] + 1e-5)
    wt = (lyr['W'] * a[:, None]).T
    b = ((lyr['b'] - lyr['mean']) * a + lyr['beta'])[None, :]
    return wt, b


def _sa_stage(xyzp, nx, xyzr, feat, layers, *, K, KC, r2, TS):
    B, S, _ = nx.shape
    N = xyzp.shape[2]
    w1t, b1 = _fold_bn(layers[0])
    w2t, b2 = _fold_bn(layers[1])
    w3t, b3 = _fold_bn(layers[2])
    w1x, w1f = w1t[:3], w1t[3:]
    cout = w3t.shape[1]

    rows = jnp.arange(N)
    m_mat = (rows[:, None] <= rows[None, :]).astype(jnp.bfloat16)
    r_mat = (jnp.arange(KC * TS)[:, None] % TS
             == jnp.arange(TS)[None, :]).astype(jnp.bfloat16)

    def full(shape):
        nd = len(shape)
        return pl.BlockSpec(shape, lambda b, s: (0,) * nd)

    in_specs = [
        pl.BlockSpec((1, 3, N), lambda b, s: (b, 0, 0)),
        pl.BlockSpec((1, TS, 3), lambda b, s: (b, s, 0)),
        full(m_mat.shape),
        full(r_mat.shape),
        pl.BlockSpec((1, N, 3), lambda b, s: (b, 0, 0)),
    ]
    args = [xyzp, nx, m_mat, r_mat, xyzr]
    if feat is not None:
        in_specs.append(pl.BlockSpec((1, N, feat.shape[2]),
                                     lambda b, s: (b, 0, 0)))
        args.append(feat)
        body = functools.partial(_sa_body_feat, K=K, KC=KC, r2=r2)
        wargs = [w1x, w1f, b1, w2t, b2, w3t, b3]
    else:
        body = functools.partial(_sa_body_nofeat, K=K, KC=KC, r2=r2)
        wargs = [w1x, b1, w2t, b2, w3t, b3]
    in_specs.extend(full(w.shape) for w in wargs)
    args.extend(wargs)

    return pl.pallas_call(
        body,
        grid=(B, S // TS),
        in_specs=in_specs,
        out_specs=pl.BlockSpec((1, TS, cout), lambda b, s: (b, s, 0)),
        out_shape=jax.ShapeDtypeStruct((B, S, cout), jnp.float32),
    )(*args)


def _sa3_body(nx_ref, f2_ref, w1x_ref, w1f_ref, b1_ref, w2_ref, b2_ref,
              w3_ref, b3_ref, out_ref):
    h = (jnp.dot(nx_ref[0], w1x_ref[...], precision=_HI)
         + jnp.dot(f2_ref[0], w1f_ref[...], precision=_HI))
    h = jnp.maximum(h + b1_ref[...], 0.0)
    h = jnp.maximum(jnp.dot(h, w2_ref[...], precision=_HI) + b2_ref[...], 0.0)
    h = jnp.maximum(jnp.dot(h, w3_ref[...], precision=_HI) + b3_ref[...], 0.0)
    out_ref[...] = jnp.max(h, axis=0, keepdims=True)[None]


def _head_body(hp_ref, wh1_ref, bh1_ref, wh2_ref, bh2_ref, wo_ref, bo_ref,
               out_ref):
    g = jnp.maximum(jnp.dot(hp_ref[...], wh1_ref[...], precision=_HI)
                    + bh1_ref[...], 0.0)
    g = jnp.maximum(jnp.dot(g, wh2_ref[...], precision=_HI) + bh2_ref[...], 0.0)
    out_ref[...] = jnp.dot(g, wo_ref[...], precision=_HI) + bo_ref[...]


def _sa3_head(nx2, f2, sa3_layers, head_layers, head_out):
    B, P, _ = nx2.shape
    F = f2.shape[2]
    w1t, b1 = _fold_bn(sa3_layers[0])
    w2t, b2 = _fold_bn(sa3_layers[1])
    w3t, b3 = _fold_bn(sa3_layers[2])
    wh1, bh1 = _fold_bn(head_layers[0])
    wh2, bh2 = _fold_bn(head_layers[1])
    wo = head_out['W'].T
    bo = head_out['b'][None, :]
    nout = wo.shape[1]
    c3 = w3t.shape[1]

    def full(shape):
        nd = len(shape)
        return pl.BlockSpec(shape, lambda b: (0,) * nd)

    hp = pl.pallas_call(
        _sa3_body,
        grid=(B,),
        in_specs=[pl.BlockSpec((1, P, 3), lambda b: (b, 0, 0)),
                  pl.BlockSpec((1, P, F), lambda b: (b, 0, 0)),
                  full(w1t[:3].shape), full(w1t[3:].shape), full(b1.shape),
                  full(w2t.shape), full(b2.shape),
                  full(w3t.shape), full(b3.shape)],
        out_specs=pl.BlockSpec((1, 1, c3), lambda b: (b, 0, 0)),
        out_shape=jax.ShapeDtypeStruct((B, 1, c3), jnp.float32),
    )(nx2, f2, w1t[:3], w1t[3:], b1, w2t, b2, w3t, b3)

    return pl.pallas_call(
        _head_body,
        out_shape=jax.ShapeDtypeStruct((B, nout), jnp.float32),
    )(hp.reshape(B, c3), wh1, bh1, wh2, bh2, wo, bo)


def kernel(pos, params):
    B, N, _ = pos.shape
    xyzp0 = jnp.transpose(pos, (2, 0, 1))
    nx1 = _fps(xyzp0, 512)
    f1 = _sa_stage(jnp.transpose(pos, (0, 2, 1)), nx1, pos, None,
                   params['sa1'], K=32, KC=16, r2=0.2 ** 2, TS=128)
    xyzp1 = jnp.transpose(nx1, (2, 0, 1))
    nx2 = _fps(xyzp1, 128)
    f2 = _sa_stage(jnp.transpose(nx1, (0, 2, 1)), nx2, nx1, f1,
                   params['sa2'], K=64, KC=16, r2=0.4 ** 2, TS=128)
    return _sa3_head(nx2, f2, params['sa3'], params['head'],
                     params['head_out'])


# speedup vs baseline: 1.2028x; 1.2028x over previous
import functools

import jax
import jax.numpy as jnp
from jax import lax
from jax.experimental import pallas as pl

_HI = lax.Precision.HIGHEST
_HG = lax.Precision.HIGH


def _fps_body(xyzp_ref, nx_ref, *, npoint):
    x = xyzp_ref[0]
    y = xyzp_ref[1]
    z = xyzp_ref[2]
    B, N = x.shape
    iota = lax.broadcasted_iota(jnp.int32, (B, N), 1)

    def body(i, carry):
        dist, far = carry
        oh = (iota == far).astype(jnp.float32)
        cx = jnp.sum(x * oh, axis=1, keepdims=True)
        cy = jnp.sum(y * oh, axis=1, keepdims=True)
        cz = jnp.sum(z * oh, axis=1, keepdims=True)
        cen = jnp.concatenate([cx, cy, cz], axis=1)
        nx_ref[:, pl.ds(i, 1), :] = cen[:, None, :]
        d = (x - cx) ** 2 + (y - cy) ** 2 + (z - cz) ** 2
        dist = jnp.minimum(dist, d)
        m = jnp.max(dist, axis=1, keepdims=True)
        far = jnp.min(jnp.where(dist == m, iota, N), axis=1, keepdims=True)
        return dist, far

    dist0 = jnp.full((B, N), 1e10, jnp.float32)
    far0 = jnp.zeros((B, 1), jnp.int32)
    lax.fori_loop(0, npoint, body, (dist0, far0))


def _fps(xyzp, npoint):
    _, B, N = xyzp.shape
    return pl.pallas_call(
        functools.partial(_fps_body, npoint=npoint),
        out_shape=jax.ShapeDtypeStruct((B, npoint, 3), jnp.float32),
    )(xyzp)


def _sa_body_nofeat(xyzp_ref, nx_ref, nxp_ref, m_ref, r_ref, xyzr_ref,
                    w1x_ref, b1_ref, w2_ref, b2_ref, w3_ref, b3_ref,
                    out_ref, *, K, KC, r2):
    _sa_common(xyzp_ref, nx_ref, nxp_ref, m_ref, r_ref, xyzr_ref, None,
               w1x_ref, None, b1_ref, w2_ref, b2_ref, w3_ref, b3_ref,
               out_ref, K=K, KC=KC, r2=r2)


def _sa_body_feat(xyzp_ref, nx_ref, nxp_ref, m_ref, r_ref, xyzr_ref, feat_ref,
                  w1x_ref, w1f_ref, b1_ref, w2_ref, b2_ref, w3_ref, b3_ref,
                  out_ref, *, K, KC, r2):
    _sa_common(xyzp_ref, nx_ref, nxp_ref, m_ref, r_ref, xyzr_ref, feat_ref,
               w1x_ref, w1f_ref, b1_ref, w2_ref, b2_ref, w3_ref, b3_ref,
               out_ref, K=K, KC=KC, r2=r2)


def _sa_common(xyzp_ref, nx_ref, nxp_ref, m_ref, r_ref, xyzr_ref, feat_ref,
               w1x_ref, w1f_ref, b1_ref, w2_ref, b2_ref, w3_ref, b3_ref,
               out_ref, *, K, KC, r2):
    xyzp = xyzp_ref[0]
    nxt = nx_ref[0]
    nxp = nxp_ref[0]
    TS = nxt.shape[0]
    N = xyzp.shape[1]
    cout = w3_ref.shape[0]
    f32 = jnp.float32
    bf16 = jnp.bfloat16

    sq = ((nxt[:, 0:1] - xyzp[0:1, :]) ** 2
          + (nxt[:, 1:2] - xyzp[1:2, :]) ** 2
          + (nxt[:, 2:3] - xyzp[2:3, :]) ** 2)
    mask = sq <= r2
    maskb = mask.astype(bf16)
    cnt = jnp.dot(maskb, m_ref[...], preferred_element_type=f32)
    cntm = cnt * mask.astype(f32)
    xyzr = xyzr_ref[0]
    xhi = xyzr.astype(bf16)
    xlo = (xyzr - xhi.astype(f32)).astype(bf16)
    nhi = nxp.astype(bf16)
    nlo = (nxp - nhi.astype(f32)).astype(bf16)
    rb = r_ref[...]
    cen = (jnp.dot(nhi, rb, preferred_element_type=f32)
           + jnp.dot(nlo, rb, preferred_element_type=f32))
    ones1n = jnp.ones((1, N), bf16)
    totalt = lax.dot_general(ones1n, maskb, (((1,), (1,)), ((), ())),
                             preferred_element_type=f32)

    kval3 = (lax.broadcasted_iota(jnp.int32, (KC, 1, 1), 0)
             .astype(f32) + 1.0)
    kval2 = lax.broadcasted_iota(jnp.int32, (KC, TS), 0).astype(f32)
    dn_t = (((0,), (1,)), ((), ()))

    def chunk(c, acc):
        base = c.astype(f32) * KC
        csh = cntm - base
        sel = csh[None, :, :] == kval3
        g = sel.astype(bf16).reshape(KC * TS, N)
        gx = (lax.dot_general(xhi, g, dn_t, preferred_element_type=f32)
              + lax.dot_general(xlo, g, dn_t, preferred_element_type=f32))
        x0 = gx - cen
        h = jnp.dot(w1x_ref[...], x0)
        if feat_ref is not None:
            gf = lax.dot_general(feat_ref[0].astype(bf16), g,
                                 (((1,), (1,)), ((), ())),
                                 preferred_element_type=f32)
            h = h + jnp.dot(w1f_ref[...], gf)
        h = jnp.maximum(h + b1_ref[...], 0.0)
        h = jnp.maximum(jnp.dot(w2_ref[...], h) + b2_ref[...], 0.0)
        h = jnp.maximum(jnp.dot(w3_ref[...], h) + b3_ref[...], 0.0)
        penalty = jnp.where((kval2 + base) < totalt, 0.0, -1e30)
        for k in range(KC):
            acc = jnp.maximum(acc, h[:, k * TS:(k + 1) * TS]
                              + penalty[k:k + 1, :])
        return acc

    acc0 = jnp.full((cout, TS), -1e30, f32)
    out_ref[0] = lax.fori_loop(0, K // KC, chunk, acc0)


def _fold_bn(lyr):
    a = lyr['gamma'] * lax.rsqrt(lyr['var'] + 1e-5)
    w = lyr['W'] * a[:, None]
    b = ((lyr['b'] - lyr['mean']) * a + lyr['beta'])[:, None]
    return w, b


def _sa_stage(xyzp, nx, nxp, xyzr, feat, layers, *, K, KC, r2, TS):
    B, S, _ = nx.shape
    N = xyzp.shape[2]
    w1, b1 = _fold_bn(layers[0])
    w2, b2 = _fold_bn(layers[1])
    w3, b3 = _fold_bn(layers[2])
    w1x, w1f = w1[:, :3], w1[:, 3:]
    cout = w3.shape[0]

    rows = jnp.arange(N)
    m_mat = (rows[:, None] <= rows[None, :]).astype(jnp.bfloat16)
    r_mat = (jnp.arange(TS)[:, None]
             == jnp.arange(KC * TS)[None, :] % TS).astype(jnp.bfloat16)

    def full(shape):
        nd = len(shape)
        return pl.BlockSpec(shape, lambda b, s: (0,) * nd)

    in_specs = [
        pl.BlockSpec((1, 3, N), lambda b, s: (b, 0, 0)),
        pl.BlockSpec((1, TS, 3), lambda b, s: (b, s, 0)),
        pl.BlockSpec((1, 3, TS), lambda b, s: (b, 0, s)),
        full(m_mat.shape),
        full(r_mat.shape),
        pl.BlockSpec((1, N, 3), lambda b, s: (b, 0, 0)),
    ]
    args = [xyzp, nx, nxp, m_mat, r_mat, xyzr]
    if feat is not None:
        in_specs.append(pl.BlockSpec((1, feat.shape[1], N),
                                     lambda b, s: (b, 0, 0)))
        args.append(feat)
        body = functools.partial(_sa_body_feat, K=K, KC=KC, r2=r2)
        wargs = [w1x, w1f, b1, w2, b2, w3, b3]
    else:
        body = functools.partial(_sa_body_nofeat, K=K, KC=KC, r2=r2)
        wargs = [w1x, b1, w2, b2, w3, b3]
    in_specs.extend(full(w.shape) for w in wargs)
    args.extend(wargs)

    return pl.pallas_call(
        body,
        grid=(B, S // TS),
        in_specs=in_specs,
        out_specs=pl.BlockSpec((1, cout, TS), lambda b, s: (b, 0, s)),
        out_shape=jax.ShapeDtypeStruct((B, cout, S), jnp.float32),
    )(*args)


def _sa3_body(nx_ref, f2_ref, w1x_ref, w1f_ref, b1_ref, w2_ref, b2_ref,
              w3_ref, b3_ref, out_ref):
    h = (jnp.dot(nx_ref[0], w1x_ref[...], precision=_HI)
         + jnp.dot(f2_ref[0], w1f_ref[...], precision=_HI))
    h = jnp.maximum(h + b1_ref[...], 0.0)
    h = jnp.maximum(jnp.dot(h, w2_ref[...], precision=_HI) + b2_ref[...], 0.0)
    h = jnp.maximum(jnp.dot(h, w3_ref[...], precision=_HI) + b3_ref[...], 0.0)
    out_ref[...] = jnp.max(h, axis=0, keepdims=True)[None]


def _head_body(hp_ref, wh1_ref, bh1_ref, wh2_ref, bh2_ref, wo_ref, bo_ref,
               out_ref):
    g = jnp.maximum(jnp.dot(hp_ref[...], wh1_ref[...], precision=_HI)
                    + bh1_ref[...], 0.0)
    g = jnp.maximum(jnp.dot(g, wh2_ref[...], precision=_HI) + bh2_ref[...], 0.0)
    out_ref[...] = jnp.dot(g, wo_ref[...], precision=_HI) + bo_ref[...]


def _sa3_head(nx2, f2, sa3_layers, head_layers, head_out):
    B, P, _ = nx2.shape
    F = f2.shape[2]

    def foldt(lyr):
        w, b = _fold_bn(lyr)
        return w.T, b.T

    w1t, b1 = foldt(sa3_layers[0])
    w2t, b2 = foldt(sa3_layers[1])
    w3t, b3 = foldt(sa3_layers[2])
    wh1, bh1 = foldt(head_layers[0])
    wh2, bh2 = foldt(head_layers[1])
    wo = head_out['W'].T
    bo = head_out['b'][None, :]
    nout = wo.shape[1]
    c3 = w3t.shape[1]

    def full(shape):
        nd = len(shape)
        return pl.BlockSpec(shape, lambda b: (0,) * nd)

    hp = pl.pallas_call(
        _sa3_body,
        grid=(B,),
        in_specs=[pl.BlockSpec((1, P, 3), lambda b: (b, 0, 0)),
                  pl.BlockSpec((1, P, F), lambda b: (b, 0, 0)),
                  full(w1t[:3].shape), full(w1t[3:].shape), full(b1.shape),
                  full(w2t.shape), full(b2.shape),
                  full(w3t.shape), full(b3.shape)],
        out_specs=pl.BlockSpec((1, 1, c3), lambda b: (b, 0, 0)),
        out_shape=jax.ShapeDtypeStruct((B, 1, c3), jnp.float32),
    )(nx2, f2, w1t[:3], w1t[3:], b1, w2t, b2, w3t, b3)

    return pl.pallas_call(
        _head_body,
        out_shape=jax.ShapeDtypeStruct((B, nout), jnp.float32),
    )(hp.reshape(B, c3), wh1, bh1, wh2, bh2, wo, bo)


def kernel(pos, params):
    B, N, _ = pos.shape
    xyzp0 = jnp.transpose(pos, (2, 0, 1))
    nx1 = _fps(xyzp0, 512)
    nxp1 = jnp.transpose(nx1, (0, 2, 1))
    f1 = _sa_stage(jnp.transpose(pos, (0, 2, 1)), nx1, nxp1, pos, None,
                   params['sa1'], K=32, KC=16, r2=0.2 ** 2, TS=128)
    nx2 = _fps(jnp.transpose(nx1, (2, 0, 1)), 128)
    nxp2 = jnp.transpose(nx2, (0, 2, 1))
    f2 = _sa_stage(nxp1, nx2, nxp2, nx1, f1,
                   params['sa2'], K=64, KC=16, r2=0.4 ** 2, TS=128)
    return _sa3_head(nx2, jnp.transpose(f2, (0, 2, 1)), params['sa3'],
                     params['head'], params['head_out'])


# speedup vs baseline: 1.5398x; 1.2801x over previous
import functools

import jax
import jax.numpy as jnp
from jax import lax
from jax.experimental import pallas as pl

_HI = lax.Precision.HIGHEST
_HG = lax.Precision.HIGH


def _fps_body(xyzp_ref, nx_ref, *, npoint):
    x = xyzp_ref[0]
    y = xyzp_ref[1]
    z = xyzp_ref[2]
    B, N = x.shape
    iota = lax.broadcasted_iota(jnp.int32, (B, N), 1)

    def body(i, carry):
        dist, far = carry
        oh = (iota == far).astype(jnp.float32)
        cx = jnp.sum(x * oh, axis=1, keepdims=True)
        cy = jnp.sum(y * oh, axis=1, keepdims=True)
        cz = jnp.sum(z * oh, axis=1, keepdims=True)
        cen = jnp.concatenate([cx, cy, cz], axis=1)
        nx_ref[:, pl.ds(i, 1), :] = cen[:, None, :]
        d = (x - cx) ** 2 + (y - cy) ** 2 + (z - cz) ** 2
        dist = jnp.minimum(dist, d)
        m = jnp.max(dist, axis=1, keepdims=True)
        far = jnp.min(jnp.where(dist == m, iota, N), axis=1, keepdims=True)
        return dist, far

    dist0 = jnp.full((B, N), 1e10, jnp.float32)
    far0 = jnp.zeros((B, 1), jnp.int32)
    lax.fori_loop(0, npoint, body, (dist0, far0))


def _fps(xyzp, npoint):
    _, B, N = xyzp.shape
    return pl.pallas_call(
        functools.partial(_fps_body, npoint=npoint),
        out_shape=jax.ShapeDtypeStruct((B, npoint, 3), jnp.float32),
    )(xyzp)


def _sa_body_nofeat(xyzp_ref, nx_ref, nxp_ref, m_ref, r_ref, xyzr_ref,
                    w1x_ref, b1_ref, w2_ref, b2_ref, w3_ref, b3_ref,
                    out_ref, *, K, KC, r2):
    _sa_common(xyzp_ref, nx_ref, nxp_ref, m_ref, r_ref, xyzr_ref, None,
               w1x_ref, None, b1_ref, w2_ref, b2_ref, w3_ref, b3_ref,
               out_ref, K=K, KC=KC, r2=r2)


def _sa_body_feat(xyzp_ref, nx_ref, nxp_ref, m_ref, r_ref, xyzr_ref, feat_ref,
                  w1x_ref, w1f_ref, b1_ref, w2_ref, b2_ref, w3_ref, b3_ref,
                  out_ref, *, K, KC, r2):
    _sa_common(xyzp_ref, nx_ref, nxp_ref, m_ref, r_ref, xyzr_ref, feat_ref,
               w1x_ref, w1f_ref, b1_ref, w2_ref, b2_ref, w3_ref, b3_ref,
               out_ref, K=K, KC=KC, r2=r2)


def _sa_common(xyzp_ref, nx_ref, nxp_ref, m_ref, r_ref, xyzr_ref, feat_ref,
               w1x_ref, w1f_ref, b1_ref, w2_ref, b2_ref, w3_ref, b3_ref,
               out_ref, *, K, KC, r2):
    xyzp = xyzp_ref[0]
    nxt = nx_ref[0]
    nxp = nxp_ref[0]
    TS = nxt.shape[0]
    N = xyzp.shape[1]
    cout = w3_ref.shape[0]
    f32 = jnp.float32
    bf16 = jnp.bfloat16

    sq = ((nxt[:, 0:1] - xyzp[0:1, :]) ** 2
          + (nxt[:, 1:2] - xyzp[1:2, :]) ** 2
          + (nxt[:, 2:3] - xyzp[2:3, :]) ** 2)
    mask = sq <= r2
    maskb = mask.astype(bf16)
    cnt = jnp.dot(maskb, m_ref[...], preferred_element_type=f32)
    cntm = cnt * mask.astype(f32)
    xyzr = xyzr_ref[0]
    xhi = xyzr.astype(bf16)
    xlo = (xyzr - xhi.astype(f32)).astype(bf16)
    xsplit = jnp.concatenate([xhi, xlo], axis=1)
    nhi = nxp.astype(bf16)
    nlo = (nxp - nhi.astype(f32)).astype(bf16)
    nsplit = jnp.concatenate([nhi, nlo], axis=0)
    rb = r_ref[...]
    cen6 = jnp.dot(nsplit, rb, preferred_element_type=f32)
    cen = cen6[0:3] + cen6[3:6]
    ones1n = jnp.ones((1, N), bf16)
    totalt = lax.dot_general(ones1n, maskb, (((1,), (1,)), ((), ())),
                             preferred_element_type=f32)

    kval3 = (lax.broadcasted_iota(jnp.int32, (KC, 1, 1), 0)
             .astype(f32) + 1.0)
    kval2 = lax.broadcasted_iota(jnp.int32, (KC, TS), 0).astype(f32)
    dn_t = (((0,), (1,)), ((), ()))

    def chunk(c, acc):
        base = c.astype(f32) * KC
        csh = cntm - base
        sel = csh[None, :, :] == kval3
        g = sel.astype(bf16).reshape(KC * TS, N)
        gx6 = lax.dot_general(xsplit, g, dn_t, preferred_element_type=f32)
        x0 = (gx6[0:3] + gx6[3:6]) - cen
        h = jnp.dot(w1x_ref[...], x0)
        if feat_ref is not None:
            gf = lax.dot_general(feat_ref[0].astype(bf16), g,
                                 (((1,), (1,)), ((), ())),
                                 preferred_element_type=f32)
            h = h + jnp.dot(w1f_ref[...], gf)
        h = jnp.maximum(h + b1_ref[...], 0.0)
        h = jnp.maximum(jnp.dot(w2_ref[...], h) + b2_ref[...], 0.0)
        h = jnp.maximum(jnp.dot(w3_ref[...], h) + b3_ref[...], 0.0)
        penalty = jnp.where((kval2 + base) < totalt, 0.0, -1e30)
        for k in range(KC):
            acc = jnp.maximum(acc, h[:, k * TS:(k + 1) * TS]
                              + penalty[k:k + 1, :])
        return acc

    acc0 = jnp.full((cout, TS), -1e30, f32)
    out_ref[0] = lax.fori_loop(0, K // KC, chunk, acc0)


def _fold_bn(lyr):
    a = lyr['gamma'] * lax.rsqrt(lyr['var'] + 1e-5)
    w = lyr['W'] * a[:, None]
    b = ((lyr['b'] - lyr['mean']) * a + lyr['beta'])[:, None]
    return w, b


def _sa_stage(xyzp, nx, nxp, xyzr, feat, layers, *, K, KC, r2, TS):
    B, S, _ = nx.shape
    N = xyzp.shape[2]
    w1, b1 = _fold_bn(layers[0])
    w2, b2 = _fold_bn(layers[1])
    w3, b3 = _fold_bn(layers[2])
    w1x, w1f = w1[:, :3], w1[:, 3:]
    cout = w3.shape[0]

    rows = jnp.arange(N)
    m_mat = (rows[:, None] <= rows[None, :]).astype(jnp.bfloat16)
    r_mat = (jnp.arange(TS)[:, None]
             == jnp.arange(KC * TS)[None, :] % TS).astype(jnp.bfloat16)

    def full(shape):
        nd = len(shape)
        return pl.BlockSpec(shape, lambda b, s: (0,) * nd)

    in_specs = [
        pl.BlockSpec((1, 3, N), lambda b, s: (b, 0, 0)),
        pl.BlockSpec((1, TS, 3), lambda b, s: (b, s, 0)),
        pl.BlockSpec((1, 3, TS), lambda b, s: (b, 0, s)),
        full(m_mat.shape),
        full(r_mat.shape),
        pl.BlockSpec((1, N, 3), lambda b, s: (b, 0, 0)),
    ]
    args = [xyzp, nx, nxp, m_mat, r_mat, xyzr]
    if feat is not None:
        in_specs.append(pl.BlockSpec((1, feat.shape[1], N),
                                     lambda b, s: (b, 0, 0)))
        args.append(feat)
        body = functools.partial(_sa_body_feat, K=K, KC=KC, r2=r2)
        wargs = [w1x, w1f, b1, w2, b2, w3, b3]
    else:
        body = functools.partial(_sa_body_nofeat, K=K, KC=KC, r2=r2)
        wargs = [w1x, b1, w2, b2, w3, b3]
    in_specs.extend(full(w.shape) for w in wargs)
    args.extend(wargs)

    return pl.pallas_call(
        body,
        grid=(B, S // TS),
        in_specs=in_specs,
        out_specs=pl.BlockSpec((1, cout, TS), lambda b, s: (b, 0, s)),
        out_shape=jax.ShapeDtypeStruct((B, cout, S), jnp.float32),
    )(*args)


def _sa3_body(nx_ref, f2_ref, w1x_ref, w1f_ref, b1_ref, w2_ref, b2_ref,
              w3_ref, b3_ref, out_ref):
    h = (jnp.dot(nx_ref[0], w1x_ref[...], precision=_HI)
         + jnp.dot(f2_ref[0], w1f_ref[...], precision=_HI))
    h = jnp.maximum(h + b1_ref[...], 0.0)
    h = jnp.maximum(jnp.dot(h, w2_ref[...], precision=_HI) + b2_ref[...], 0.0)
    h = jnp.maximum(jnp.dot(h, w3_ref[...], precision=_HI) + b3_ref[...], 0.0)
    out_ref[...] = jnp.max(h, axis=0, keepdims=True)[None]


def _head_body(hp_ref, wh1_ref, bh1_ref, wh2_ref, bh2_ref, wo_ref, bo_ref,
               out_ref):
    g = jnp.maximum(jnp.dot(hp_ref[...], wh1_ref[...], precision=_HI)
                    + bh1_ref[...], 0.0)
    g = jnp.maximum(jnp.dot(g, wh2_ref[...], precision=_HI) + bh2_ref[...], 0.0)
    out_ref[...] = jnp.dot(g, wo_ref[...], precision=_HI) + bo_ref[...]


def _sa3_head(nx2, f2, sa3_layers, head_layers, head_out):
    B, P, _ = nx2.shape
    F = f2.shape[2]

    def foldt(lyr):
        w, b = _fold_bn(lyr)
        return w.T, b.T

    w1t, b1 = foldt(sa3_layers[0])
    w2t, b2 = foldt(sa3_layers[1])
    w3t, b3 = foldt(sa3_layers[2])
    wh1, bh1 = foldt(head_layers[0])
    wh2, bh2 = foldt(head_layers[1])
    wo = head_out['W'].T
    bo = head_out['b'][None, :]
    nout = wo.shape[1]
    c3 = w3t.shape[1]

    def full(shape):
        nd = len(shape)
        return pl.BlockSpec(shape, lambda b: (0,) * nd)

    hp = pl.pallas_call(
        _sa3_body,
        grid=(B,),
        in_specs=[pl.BlockSpec((1, P, 3), lambda b: (b, 0, 0)),
                  pl.BlockSpec((1, P, F), lambda b: (b, 0, 0)),
                  full(w1t[:3].shape), full(w1t[3:].shape), full(b1.shape),
                  full(w2t.shape), full(b2.shape),
                  full(w3t.shape), full(b3.shape)],
        out_specs=pl.BlockSpec((1, 1, c3), lambda b: (b, 0, 0)),
        out_shape=jax.ShapeDtypeStruct((B, 1, c3), jnp.float32),
    )(nx2, f2, w1t[:3], w1t[3:], b1, w2t, b2, w3t, b3)

    return pl.pallas_call(
        _head_body,
        out_shape=jax.ShapeDtypeStruct((B, nout), jnp.float32),
    )(hp.reshape(B, c3), wh1, bh1, wh2, bh2, wo, bo)


def kernel(pos, params):
    B, N, _ = pos.shape
    xyzp0 = jnp.transpose(pos, (2, 0, 1))
    nx1 = _fps(xyzp0, 512)
    nxp1 = jnp.transpose(nx1, (0, 2, 1))
    f1 = _sa_stage(jnp.transpose(pos, (0, 2, 1)), nx1, nxp1, pos, None,
                   params['sa1'], K=32, KC=16, r2=0.2 ** 2, TS=128)
    nx2 = _fps(jnp.transpose(nx1, (2, 0, 1)), 128)
    nxp2 = jnp.transpose(nx2, (0, 2, 1))
    f2 = _sa_stage(nxp1, nx2, nxp2, nx1, f1,
                   params['sa2'], K=64, KC=16, r2=0.4 ** 2, TS=128)
    return _sa3_head(nx2, jnp.transpose(f2, (0, 2, 1)), params['sa3'],
                     params['head'], params['head_out'])


# speedup vs baseline: 1.6659x; 1.0819x over previous
import functools

import jax
import jax.numpy as jnp
from jax import lax
from jax.experimental import pallas as pl

_HI = lax.Precision.HIGHEST
_HG = lax.Precision.HIGH


def _fps_body(xyzp_ref, nx_ref, *, npoint):
    x = xyzp_ref[0]
    y = xyzp_ref[1]
    z = xyzp_ref[2]
    B, N = x.shape
    iota = lax.broadcasted_iota(jnp.int32, (B, N), 1)

    def body(i, carry):
        dist, far = carry
        oh = (iota == far).astype(jnp.float32)
        cx = jnp.sum(x * oh, axis=1, keepdims=True)
        cy = jnp.sum(y * oh, axis=1, keepdims=True)
        cz = jnp.sum(z * oh, axis=1, keepdims=True)
        cen = jnp.concatenate([cx, cy, cz], axis=1)
        nx_ref[:, pl.ds(i, 1), :] = cen[:, None, :]
        d = (x - cx) ** 2 + (y - cy) ** 2 + (z - cz) ** 2
        dist = jnp.minimum(dist, d)
        m = jnp.max(dist, axis=1, keepdims=True)
        far = jnp.min(jnp.where(dist == m, iota, N), axis=1, keepdims=True)
        return dist, far

    dist0 = jnp.full((B, N), 1e10, jnp.float32)
    far0 = jnp.zeros((B, 1), jnp.int32)
    lax.fori_loop(0, npoint, body, (dist0, far0))


def _fps(xyzp, npoint):
    _, B, N = xyzp.shape
    return pl.pallas_call(
        functools.partial(_fps_body, npoint=npoint),
        out_shape=jax.ShapeDtypeStruct((B, npoint, 3), jnp.float32),
    )(xyzp)


def _sa_body_nofeat(xyzp_ref, nx_ref, nxp_ref, m_ref, r_ref, xyzr_ref,
                    w1x_ref, b1_ref, w2_ref, b2_ref, w3_ref, b3_ref,
                    out_ref, *, K, KC, r2):
    _sa_common(xyzp_ref, nx_ref, nxp_ref, m_ref, r_ref, xyzr_ref, None,
               w1x_ref, None, b1_ref, w2_ref, b2_ref, w3_ref, b3_ref,
               out_ref, K=K, KC=KC, r2=r2)


def _sa_body_feat(xyzp_ref, nx_ref, nxp_ref, m_ref, r_ref, xyzr_ref, feat_ref,
                  w1x_ref, w1f_ref, b1_ref, w2_ref, b2_ref, w3_ref, b3_ref,
                  out_ref, *, K, KC, r2):
    _sa_common(xyzp_ref, nx_ref, nxp_ref, m_ref, r_ref, xyzr_ref, feat_ref,
               w1x_ref, w1f_ref, b1_ref, w2_ref, b2_ref, w3_ref, b3_ref,
               out_ref, K=K, KC=KC, r2=r2)


def _sa_common(xyzp_ref, nx_ref, nxp_ref, m_ref, r_ref, xyzr_ref, feat_ref,
               w1x_ref, w1f_ref, b1_ref, w2_ref, b2_ref, w3_ref, b3_ref,
               out_ref, *, K, KC, r2):
    xyzp = xyzp_ref[0]
    nxt = nx_ref[0]
    nxp = nxp_ref[0]
    TS = nxt.shape[0]
    N = xyzp.shape[1]
    cout = w3_ref.shape[0]
    f32 = jnp.float32
    bf16 = jnp.bfloat16

    sq = ((nxt[:, 0:1] - xyzp[0:1, :]) ** 2
          + (nxt[:, 1:2] - xyzp[1:2, :]) ** 2
          + (nxt[:, 2:3] - xyzp[2:3, :]) ** 2)
    mask = sq <= r2
    maskb = mask.astype(bf16)
    cnt = jnp.dot(maskb, m_ref[...], preferred_element_type=f32)
    cntm = cnt * mask.astype(f32)
    xyzr = xyzr_ref[0]
    xhi = xyzr.astype(bf16)
    xlo = (xyzr - xhi.astype(f32)).astype(bf16)
    xsplit = jnp.concatenate([xhi, xlo], axis=1)
    nhi = nxp.astype(bf16)
    nlo = (nxp - nhi.astype(f32)).astype(bf16)
    nsplit = jnp.concatenate([nhi, nlo], axis=0)
    rb = r_ref[...]
    cen6 = jnp.dot(nsplit, rb, preferred_element_type=f32)
    cen = cen6[0:3] + cen6[3:6]
    ones1n = jnp.ones((1, N), bf16)
    totalt = lax.dot_general(ones1n, maskb, (((1,), (1,)), ((), ())),
                             preferred_element_type=f32)

    kval3 = (lax.broadcasted_iota(jnp.int32, (KC, 1, 1), 0)
             .astype(f32) + 1.0)
    kval2 = lax.broadcasted_iota(jnp.int32, (KC, TS), 0).astype(f32)
    dn_t = (((0,), (1,)), ((), ()))

    def chunk(c, acc):
        base = c.astype(f32) * KC
        csh = cntm - base
        sel = csh[None, :, :] == kval3
        g = sel.astype(bf16).reshape(KC * TS, N)
        gx6 = lax.dot_general(xsplit, g, dn_t, preferred_element_type=f32)
        x0 = (gx6[0:3] + gx6[3:6]) - cen
        h = jnp.dot(w1x_ref[...], x0)
        if feat_ref is not None:
            gf = lax.dot_general(feat_ref[0].astype(bf16), g,
                                 (((1,), (1,)), ((), ())),
                                 preferred_element_type=f32)
            h = h + jnp.dot(w1f_ref[...], gf)
        h = jnp.maximum(h + b1_ref[...], 0.0)
        h = jnp.maximum(jnp.dot(w2_ref[...], h) + b2_ref[...], 0.0)
        h = jnp.maximum(jnp.dot(w3_ref[...], h) + b3_ref[...], 0.0)
        penalty = jnp.where((kval2 + base) < totalt, 0.0, -1e30)
        for k in range(KC):
            acc = jnp.maximum(acc, h[:, k * TS:(k + 1) * TS]
                              + penalty[k:k + 1, :])
        return acc

    acc0 = jnp.full((cout, TS), -1e30, f32)
    out_ref[0] = lax.fori_loop(0, K // KC, chunk, acc0)


def _fold_bn(lyr):
    a = lyr['gamma'] * lax.rsqrt(lyr['var'] + 1e-5)
    w = lyr['W'] * a[:, None]
    b = ((lyr['b'] - lyr['mean']) * a + lyr['beta'])[:, None]
    return w, b


def _sa_stage(xyzp, nx, nxp, xyzr, feat, layers, *, K, KC, r2, TS):
    B, S, _ = nx.shape
    N = xyzp.shape[2]
    w1, b1 = _fold_bn(layers[0])
    w2, b2 = _fold_bn(layers[1])
    w3, b3 = _fold_bn(layers[2])
    w1x, w1f = w1[:, :3], w1[:, 3:]
    cout = w3.shape[0]

    rows = jnp.arange(N)
    m_mat = (rows[:, None] <= rows[None, :]).astype(jnp.bfloat16)
    r_mat = (jnp.arange(TS)[:, None]
             == jnp.arange(KC * TS)[None, :] % TS).astype(jnp.bfloat16)

    def full(shape):
        nd = len(shape)
        return pl.BlockSpec(shape, lambda b, s: (0,) * nd)

    in_specs = [
        pl.BlockSpec((1, 3, N), lambda b, s: (b, 0, 0)),
        pl.BlockSpec((1, TS, 3), lambda b, s: (b, s, 0)),
        pl.BlockSpec((1, 3, TS), lambda b, s: (b, 0, s)),
        full(m_mat.shape),
        full(r_mat.shape),
        pl.BlockSpec((1, N, 3), lambda b, s: (b, 0, 0)),
    ]
    args = [xyzp, nx, nxp, m_mat, r_mat, xyzr]
    if feat is not None:
        in_specs.append(pl.BlockSpec((1, feat.shape[1], N),
                                     lambda b, s: (b, 0, 0)))
        args.append(feat)
        body = functools.partial(_sa_body_feat, K=K, KC=KC, r2=r2)
        wargs = [w1x, w1f, b1, w2, b2, w3, b3]
    else:
        body = functools.partial(_sa_body_nofeat, K=K, KC=KC, r2=r2)
        wargs = [w1x, b1, w2, b2, w3, b3]
    in_specs.extend(full(w.shape) for w in wargs)
    args.extend(wargs)

    return pl.pallas_call(
        body,
        grid=(B, S // TS),
        in_specs=in_specs,
        out_specs=pl.BlockSpec((1, cout, TS), lambda b, s: (b, 0, s)),
        out_shape=jax.ShapeDtypeStruct((B, cout, S), jnp.float32),
    )(*args)


def _sa3_body(nx_ref, f2_ref, w1x_ref, w1f_ref, b1_ref, w2_ref, b2_ref,
              w3_ref, b3_ref, out_ref):
    h = (jnp.dot(nx_ref[0], w1x_ref[...], precision=_HI)
         + jnp.dot(f2_ref[0], w1f_ref[...], precision=_HI))
    h = jnp.maximum(h + b1_ref[...], 0.0)
    h = jnp.maximum(jnp.dot(h, w2_ref[...], precision=_HI) + b2_ref[...], 0.0)
    h = jnp.maximum(jnp.dot(h, w3_ref[...], precision=_HI) + b3_ref[...], 0.0)
    out_ref[...] = jnp.max(h, axis=0, keepdims=True)[None]


def _head_body(hp_ref, wh1_ref, bh1_ref, wh2_ref, bh2_ref, wo_ref, bo_ref,
               out_ref):
    g = jnp.maximum(jnp.dot(hp_ref[...], wh1_ref[...], precision=_HI)
                    + bh1_ref[...], 0.0)
    g = jnp.maximum(jnp.dot(g, wh2_ref[...], precision=_HI) + bh2_ref[...], 0.0)
    out_ref[...] = jnp.dot(g, wo_ref[...], precision=_HI) + bo_ref[...]


def _sa3_head(nx2, f2, sa3_layers, head_layers, head_out):
    B, P, _ = nx2.shape
    F = f2.shape[2]

    def foldt(lyr):
        w, b = _fold_bn(lyr)
        return w.T, b.T

    w1t, b1 = foldt(sa3_layers[0])
    w2t, b2 = foldt(sa3_layers[1])
    w3t, b3 = foldt(sa3_layers[2])
    wh1, bh1 = foldt(head_layers[0])
    wh2, bh2 = foldt(head_layers[1])
    wo = head_out['W'].T
    bo = head_out['b'][None, :]
    nout = wo.shape[1]
    c3 = w3t.shape[1]

    def full(shape):
        nd = len(shape)
        return pl.BlockSpec(shape, lambda b: (0,) * nd)

    hp = pl.pallas_call(
        _sa3_body,
        grid=(B,),
        in_specs=[pl.BlockSpec((1, P, 3), lambda b: (b, 0, 0)),
                  pl.BlockSpec((1, P, F), lambda b: (b, 0, 0)),
                  full(w1t[:3].shape), full(w1t[3:].shape), full(b1.shape),
                  full(w2t.shape), full(b2.shape),
                  full(w3t.shape), full(b3.shape)],
        out_specs=pl.BlockSpec((1, 1, c3), lambda b: (b, 0, 0)),
        out_shape=jax.ShapeDtypeStruct((B, 1, c3), jnp.float32),
    )(nx2, f2, w1t[:3], w1t[3:], b1, w2t, b2, w3t, b3)

    return pl.pallas_call(
        _head_body,
        out_shape=jax.ShapeDtypeStruct((B, nout), jnp.float32),
    )(hp.reshape(B, c3), wh1, bh1, wh2, bh2, wo, bo)


def kernel(pos, params):
    B, N, _ = pos.shape
    xyzp0 = jnp.transpose(pos, (2, 0, 1))
    nx1 = _fps(xyzp0, 512)
    nxp1 = jnp.transpose(nx1, (0, 2, 1))
    f1 = _sa_stage(jnp.transpose(pos, (0, 2, 1)), nx1, nxp1, pos, None,
                   params['sa1'], K=32, KC=32, r2=0.2 ** 2, TS=128)
    nx2 = _fps(jnp.transpose(nx1, (2, 0, 1)), 128)
    nxp2 = jnp.transpose(nx2, (0, 2, 1))
    f2 = _sa_stage(nxp1, nx2, nxp2, nx1, f1,
                   params['sa2'], K=64, KC=64, r2=0.4 ** 2, TS=128)
    return _sa3_head(nx2, jnp.transpose(f2, (0, 2, 1)), params['sa3'],
                     params['head'], params['head_out'])


# speedup vs baseline: 1.6928x; 1.0161x over previous
import functools

import jax
import jax.numpy as jnp
from jax import lax
from jax.experimental import pallas as pl

_HI = lax.Precision.HIGHEST
_HG = lax.Precision.HIGH


def _fps_body(xyzp_ref, nx_ref, *, npoint):
    x = xyzp_ref[0]
    y = xyzp_ref[1]
    z = xyzp_ref[2]
    B, N = x.shape
    iota = lax.broadcasted_iota(jnp.int32, (B, N), 1)

    def body(i, carry):
        dist, far = carry
        oh = (iota == far).astype(jnp.float32)
        cx = jnp.sum(x * oh, axis=1, keepdims=True)
        cy = jnp.sum(y * oh, axis=1, keepdims=True)
        cz = jnp.sum(z * oh, axis=1, keepdims=True)
        cen = jnp.concatenate([cx, cy, cz], axis=1)
        nx_ref[:, pl.ds(i, 1), :] = cen[:, None, :]
        d = (x - cx) ** 2 + (y - cy) ** 2 + (z - cz) ** 2
        dist = jnp.minimum(dist, d)
        m = jnp.max(dist, axis=1, keepdims=True)
        far = jnp.min(jnp.where(dist == m, iota, N), axis=1, keepdims=True)
        return dist, far

    dist0 = jnp.full((B, N), 1e10, jnp.float32)
    far0 = jnp.zeros((B, 1), jnp.int32)
    lax.fori_loop(0, npoint, body, (dist0, far0))


def _fps(xyzp, npoint):
    _, B, N = xyzp.shape
    return pl.pallas_call(
        functools.partial(_fps_body, npoint=npoint),
        out_shape=jax.ShapeDtypeStruct((B, npoint, 3), jnp.float32),
    )(xyzp)


def _sa_body_nofeat(xyzp_ref, nx_ref, nxp_ref, m_ref, r_ref, xyzr_ref,
                    w1x_ref, b1_ref, w2_ref, b2_ref, w3_ref, b3_ref,
                    out_ref, *, K, KC, r2):
    _sa_common(xyzp_ref, nx_ref, nxp_ref, m_ref, r_ref, xyzr_ref, None,
               w1x_ref, None, b1_ref, w2_ref, b2_ref, w3_ref, b3_ref,
               out_ref, K=K, KC=KC, r2=r2)


def _sa_body_feat(xyzp_ref, nx_ref, nxp_ref, m_ref, r_ref, xyzr_ref, feat_ref,
                  w1x_ref, w1f_ref, b1_ref, w2_ref, b2_ref, w3_ref, b3_ref,
                  out_ref, *, K, KC, r2):
    _sa_common(xyzp_ref, nx_ref, nxp_ref, m_ref, r_ref, xyzr_ref, feat_ref,
               w1x_ref, w1f_ref, b1_ref, w2_ref, b2_ref, w3_ref, b3_ref,
               out_ref, K=K, KC=KC, r2=r2)


def _sa_common(xyzp_ref, nx_ref, nxp_ref, m_ref, r_ref, xyzr_ref, feat_ref,
               w1x_ref, w1f_ref, b1_ref, w2_ref, b2_ref, w3_ref, b3_ref,
               out_ref, *, K, KC, r2):
    xyzp = xyzp_ref[0]
    nxt = nx_ref[0]
    nxp = nxp_ref[0]
    TS = nxt.shape[0]
    N = xyzp.shape[1]
    cout = w3_ref.shape[0]
    f32 = jnp.float32
    bf16 = jnp.bfloat16

    sq = ((nxt[:, 0:1] - xyzp[0:1, :]) ** 2
          + (nxt[:, 1:2] - xyzp[1:2, :]) ** 2
          + (nxt[:, 2:3] - xyzp[2:3, :]) ** 2)
    mask = sq <= r2
    maskb = mask.astype(bf16)
    cnt = jnp.dot(maskb, m_ref[...], preferred_element_type=f32)
    cntm = cnt * mask.astype(f32)
    xyzr = xyzr_ref[0]
    xhi = xyzr.astype(bf16)
    xlo = (xyzr - xhi.astype(f32)).astype(bf16)
    xsplit = jnp.concatenate([xhi, xlo], axis=1)
    nhi = nxp.astype(bf16)
    nlo = (nxp - nhi.astype(f32)).astype(bf16)
    nsplit = jnp.concatenate([nhi, nlo], axis=0)
    rb = r_ref[...]
    cen6 = jnp.dot(nsplit, rb, preferred_element_type=f32)
    cen = cen6[0:3] + cen6[3:6]
    ones1n = jnp.ones((1, N), bf16)
    totalt = lax.dot_general(ones1n, maskb, (((1,), (1,)), ((), ())),
                             preferred_element_type=f32)

    kval3 = (lax.broadcasted_iota(jnp.int32, (KC, 1, 1), 0)
             .astype(f32) + 1.0)
    kval2 = lax.broadcasted_iota(jnp.int32, (KC, TS), 0).astype(f32)
    dn_t = (((0,), (1,)), ((), ()))

    def chunk(c, acc):
        base = c.astype(f32) * KC
        csh = cntm - base
        sel = csh[None, :, :] == kval3
        g = sel.astype(bf16).reshape(KC * TS, N)
        gx6 = lax.dot_general(xsplit, g, dn_t, preferred_element_type=f32)
        x0 = (gx6[0:3] + gx6[3:6]) - cen
        h = jnp.dot(w1x_ref[...], x0)
        if feat_ref is not None:
            gf = lax.dot_general(feat_ref[0].astype(bf16), g,
                                 (((1,), (1,)), ((), ())),
                                 preferred_element_type=f32)
            h = h + jnp.dot(w1f_ref[...], gf)
        h = jnp.maximum(h + b1_ref[...], 0.0)
        h = jnp.maximum(jnp.dot(w2_ref[...], h) + b2_ref[...], 0.0)
        h = jnp.maximum(jnp.dot(w3_ref[...], h) + b3_ref[...], 0.0)
        penalty = jnp.where((kval2 + base) < totalt, 0.0, -1e30)
        for k in range(KC):
            acc = jnp.maximum(acc, h[:, k * TS:(k + 1) * TS]
                              + penalty[k:k + 1, :])
        return acc

    acc0 = jnp.full((cout, TS), -1e30, f32)
    out_ref[0] = lax.fori_loop(0, K // KC, chunk, acc0)


def _fold_bn(lyr):
    a = lyr['gamma'] * lax.rsqrt(lyr['var'] + 1e-5)
    w = lyr['W'] * a[:, None]
    b = ((lyr['b'] - lyr['mean']) * a + lyr['beta'])[:, None]
    return w, b


def _sa_stage(xyzp, nx, nxp, xyzr, feat, layers, *, K, KC, r2, TS):
    B, S, _ = nx.shape
    N = xyzp.shape[2]
    w1, b1 = _fold_bn(layers[0])
    w2, b2 = _fold_bn(layers[1])
    w3, b3 = _fold_bn(layers[2])
    w1x, w1f = w1[:, :3], w1[:, 3:]
    cout = w3.shape[0]

    rows = jnp.arange(N)
    m_mat = (rows[:, None] <= rows[None, :]).astype(jnp.bfloat16)
    r_mat = (jnp.arange(TS)[:, None]
             == jnp.arange(KC * TS)[None, :] % TS).astype(jnp.bfloat16)

    def full(shape):
        nd = len(shape)
        return pl.BlockSpec(shape, lambda b, s: (0,) * nd)

    in_specs = [
        pl.BlockSpec((1, 3, N), lambda b, s: (b, 0, 0)),
        pl.BlockSpec((1, TS, 3), lambda b, s: (b, s, 0)),
        pl.BlockSpec((1, 3, TS), lambda b, s: (b, 0, s)),
        full(m_mat.shape),
        full(r_mat.shape),
        pl.BlockSpec((1, N, 3), lambda b, s: (b, 0, 0)),
    ]
    args = [xyzp, nx, nxp, m_mat, r_mat, xyzr]
    if feat is not None:
        in_specs.append(pl.BlockSpec((1, feat.shape[1], N),
                                     lambda b, s: (b, 0, 0)))
        args.append(feat)
        body = functools.partial(_sa_body_feat, K=K, KC=KC, r2=r2)
        wargs = [w1x, w1f, b1, w2, b2, w3, b3]
    else:
        body = functools.partial(_sa_body_nofeat, K=K, KC=KC, r2=r2)
        wargs = [w1x, b1, w2, b2, w3, b3]
    in_specs.extend(full(w.shape) for w in wargs)
    args.extend(wargs)

    return pl.pallas_call(
        body,
        grid=(B, S // TS),
        in_specs=in_specs,
        out_specs=pl.BlockSpec((1, cout, TS), lambda b, s: (b, 0, s)),
        out_shape=jax.ShapeDtypeStruct((B, cout, S), jnp.float32),
    )(*args)


def _sa3_body(nx_ref, f2_ref, w1x_ref, w1f_ref, b1_ref, w2_ref, b2_ref,
              w3_ref, b3_ref, out_ref, *, BB, P):
    nx = nx_ref[...].reshape(BB * P, 3)
    f2 = f2_ref[...].reshape(BB * P, f2_ref.shape[2])
    h = (jnp.dot(nx, w1x_ref[...], precision=_HI)
         + jnp.dot(f2, w1f_ref[...], precision=_HI))
    h = jnp.maximum(h + b1_ref[...], 0.0)
    h = jnp.maximum(jnp.dot(h, w2_ref[...], precision=_HI) + b2_ref[...], 0.0)
    h = jnp.maximum(jnp.dot(h, w3_ref[...], precision=_HI) + b3_ref[...], 0.0)
    out_ref[...] = jnp.max(h.reshape(BB, P, h.shape[1]), axis=1,
                           keepdims=True)


def _head_body(hp_ref, wh1_ref, bh1_ref, wh2_ref, bh2_ref, wo_ref, bo_ref,
               out_ref):
    g = jnp.maximum(jnp.dot(hp_ref[...], wh1_ref[...], precision=_HI)
                    + bh1_ref[...], 0.0)
    g = jnp.maximum(jnp.dot(g, wh2_ref[...], precision=_HI) + bh2_ref[...], 0.0)
    out_ref[...] = jnp.dot(g, wo_ref[...], precision=_HI) + bo_ref[...]


def _sa3_head(nx2, f2, sa3_layers, head_layers, head_out):
    B, P, _ = nx2.shape
    F = f2.shape[2]

    def foldt(lyr):
        w, b = _fold_bn(lyr)
        return w.T, b.T

    w1t, b1 = foldt(sa3_layers[0])
    w2t, b2 = foldt(sa3_layers[1])
    w3t, b3 = foldt(sa3_layers[2])
    wh1, bh1 = foldt(head_layers[0])
    wh2, bh2 = foldt(head_layers[1])
    wo = head_out['W'].T
    bo = head_out['b'][None, :]
    nout = wo.shape[1]
    c3 = w3t.shape[1]

    def full(shape):
        nd = len(shape)
        return pl.BlockSpec(shape, lambda b: (0,) * nd)

    BB = 8
    hp = pl.pallas_call(
        functools.partial(_sa3_body, BB=BB, P=P),
        grid=(B // BB,),
        in_specs=[pl.BlockSpec((BB, P, 3), lambda b: (b, 0, 0)),
                  pl.BlockSpec((BB, P, F), lambda b: (b, 0, 0)),
                  full(w1t[:3].shape), full(w1t[3:].shape), full(b1.shape),
                  full(w2t.shape), full(b2.shape),
                  full(w3t.shape), full(b3.shape)],
        out_specs=pl.BlockSpec((BB, 1, c3), lambda b: (b, 0, 0)),
        out_shape=jax.ShapeDtypeStruct((B, 1, c3), jnp.float32),
    )(nx2, f2, w1t[:3], w1t[3:], b1, w2t, b2, w3t, b3)

    return pl.pallas_call(
        _head_body,
        out_shape=jax.ShapeDtypeStruct((B, nout), jnp.float32),
    )(hp.reshape(B, c3), wh1, bh1, wh2, bh2, wo, bo)


def kernel(pos, params):
    B, N, _ = pos.shape
    xyzp0 = jnp.transpose(pos, (2, 0, 1))
    nx1 = _fps(xyzp0, 512)
    nxp1 = jnp.transpose(nx1, (0, 2, 1))
    f1 = _sa_stage(jnp.transpose(pos, (0, 2, 1)), nx1, nxp1, pos, None,
                   params['sa1'], K=32, KC=32, r2=0.2 ** 2, TS=256)
    nx2 = _fps(jnp.transpose(nx1, (2, 0, 1)), 128)
    nxp2 = jnp.transpose(nx2, (0, 2, 1))
    f2 = _sa_stage(nxp1, nx2, nxp2, nx1, f1,
                   params['sa2'], K=64, KC=64, r2=0.4 ** 2, TS=128)
    return _sa3_head(nx2, jnp.transpose(f2, (0, 2, 1)), params['sa3'],
                     params['head'], params['head_out'])
